# Initial kernel scaffold; baseline (speedup 1.0000x reference)
import functools

import jax
import jax.numpy as jnp
from jax import lax
from jax.experimental import pallas as pl
from jax.experimental.pallas import tpu as pltpu
from jax.experimental.pallas import tpu_sc as plsc

_CHUNK = 128


@functools.cache
def _build(B, V, D, idx_dtype):
    info = plsc.get_sparse_core_info()
    nc, ns = info.num_cores, info.num_subcores
    nw = nc * ns
    assert B % (nw * _CHUNK) == 0
    b_per_w = B // nw
    n_chunks = b_per_w // _CHUNK

    mesh = plsc.VectorSubcoreMesh(core_axis_name="c", subcore_axis_name="s")

    @functools.partial(
        pl.kernel,
        mesh=mesh,
        out_type=jax.ShapeDtypeStruct((B, D), jnp.float32),
        scratch_types=[
            pltpu.VMEM((n_chunks, _CHUNK), jnp.int32),
            pltpu.VMEM((n_chunks, _CHUNK, D), jnp.float32),
            pltpu.SemaphoreType.DMA,
            pltpu.SemaphoreType.DMA,
        ],
    )
    def emb(idx_hbm, table_hbm, out_hbm, idx_v, rows_v, gsem, ssem):
        wid = lax.axis_index("s") * nc + lax.axis_index("c")
        base = wid * b_per_w
        pltpu.sync_copy(idx_hbm.at[wid], idx_v)
        gathers = []
        for j in range(n_chunks):
            gathers.append(
                pltpu.async_copy(table_hbm.at[idx_v.at[j]], rows_v.at[j], gsem)
            )
        stores = []
        for j in range(n_chunks):
            gathers[j].wait()
            stores.append(
                pltpu.async_copy(
                    rows_v.at[j], out_hbm.at[pl.ds(base + j * _CHUNK, _CHUNK)], ssem
                )
            )
        for s in stores:
            s.wait()

    def run(world_id, table):
        idx3 = world_id.reshape(nw, n_chunks, _CHUNK)
        return emb(idx3, table)

    return run


def kernel(world_id, table):
    B, = world_id.shape
    V, D = table.shape
    return _build(B, V, D, world_id.dtype.name)(world_id, table)


# speedup vs baseline: 1.6600x; 1.6600x over previous
import functools

import jax
import jax.numpy as jnp
from jax import lax
from jax.experimental import pallas as pl
from jax.experimental.pallas import tpu as pltpu
from jax.experimental.pallas import tpu_sc as plsc

_CHUNK = 128


@functools.cache
def _build(B, V, D, idx_dtype):
    info = plsc.get_sparse_core_info()
    nc, ns = info.num_cores, info.num_subcores
    nw = nc * ns
    assert B % (nw * _CHUNK) == 0
    b_per_w = B // nw
    n_chunks = b_per_w // _CHUNK

    mesh = plsc.VectorSubcoreMesh(core_axis_name="c", subcore_axis_name="s")

    @functools.partial(
        pl.kernel,
        mesh=mesh,
        out_type=jax.ShapeDtypeStruct((B, D), jnp.float32),
        scratch_types=[
            pltpu.VMEM((n_chunks, _CHUNK), jnp.int32),
            pltpu.VMEM((n_chunks, _CHUNK, D), jnp.float32),
            pltpu.SemaphoreType.DMA,
            pltpu.SemaphoreType.DMA,
        ],
        compiler_params=pltpu.CompilerParams(use_tc_tiling_on_sc=False),
    )
    def emb(idx_hbm, table_hbm, out_hbm, idx_v, rows_v, gsem, ssem):
        wid = lax.axis_index("s") * nc + lax.axis_index("c")
        base = wid * b_per_w
        pltpu.sync_copy(idx_hbm.at[wid], idx_v)
        gathers = []
        for j in range(n_chunks):
            gathers.append(
                pltpu.async_copy(table_hbm.at[idx_v.at[j]], rows_v.at[j], gsem)
            )
        stores = []
        for j in range(n_chunks):
            gathers[j].wait()
            stores.append(
                pltpu.async_copy(
                    rows_v.at[j], out_hbm.at[pl.ds(base + j * _CHUNK, _CHUNK)], ssem
                )
            )
        for s in stores:
            s.wait()

    def run(world_id, table):
        idx3 = world_id.reshape(nw, n_chunks, _CHUNK)
        return emb(idx3, table)

    return run


def kernel(world_id, table):
    B, = world_id.shape
    V, D = table.shape
    return _build(B, V, D, world_id.dtype.name)(world_id, table)


# speedup vs baseline: 1.8822x; 1.1339x over previous
import functools

import jax
import jax.numpy as jnp
from jax import lax
from jax.experimental import pallas as pl
from jax.experimental.pallas import tpu as pltpu
from jax.experimental.pallas import tpu_sc as plsc

_CHUNK = 128
_LANES = 16


@functools.cache
def _build(B, V, D):
    info = plsc.get_sparse_core_info()
    nc, ns = info.num_cores, info.num_subcores
    nw = nc * ns
    assert B % (nw * _CHUNK) == 0
    b_per_w = B // nw
    n_chunks = b_per_w // _CHUNK
    groups = _CHUNK // _LANES

    mesh = plsc.VectorSubcoreMesh(core_axis_name="c", subcore_axis_name="s")

    @functools.partial(
        pl.kernel,
        mesh=mesh,
        out_type=jax.ShapeDtypeStruct((D, B), jnp.float32),
        scratch_types=[
            pltpu.VMEM((b_per_w,), jnp.int32),
            pltpu.VMEM((V, D), jnp.float32),
            pltpu.VMEM((n_chunks, D, _CHUNK), jnp.float32),
            pltpu.SemaphoreType.DMA,
        ],
        compiler_params=pltpu.CompilerParams(needs_layout_passes=False),
    )
    def emb(idx_hbm, table_hbm, out_hbm, idx_v, table_v, buf, ssem):
        wid = lax.axis_index("s") * nc + lax.axis_index("c")
        base = wid * b_per_w
        pltpu.sync_copy(idx_hbm.at[pl.ds(base, b_per_w)], idx_v)
        pltpu.sync_copy(table_hbm, table_v)
        stores = []
        for j in range(n_chunks):
            for g in range(groups):
                idxv = idx_v[pl.ds(j * _CHUNK + g * _LANES, _LANES)]
                for d in range(D):
                    col = jnp.full((_LANES,), d, jnp.int32)
                    v = plsc.load_gather(table_v, [idxv, col])
                    buf[j, d, pl.ds(g * _LANES, _LANES)] = v
            stores.append(
                pltpu.async_copy(
                    buf.at[j], out_hbm.at[:, pl.ds(base + j * _CHUNK, _CHUNK)], ssem
                )
            )
        for s in stores:
            s.wait()

    def run(world_id, table):
        return emb(world_id, table).T

    return run


def kernel(world_id, table):
    B, = world_id.shape
    V, D = table.shape
    return _build(B, V, D)(world_id, table)


# speedup vs baseline: 2.5249x; 1.3415x over previous
import functools

import jax
import jax.numpy as jnp
from jax import lax
from jax.experimental import pallas as pl
from jax.experimental.pallas import tpu as pltpu
from jax.experimental.pallas import tpu_sc as plsc

_LANES = 16


@functools.cache
def _build(B, V, D):
    info = plsc.get_sparse_core_info()
    nc, ns = info.num_cores, info.num_subcores
    nw = nc * ns
    assert B % (nw * _LANES) == 0
    b_per_w = B // nw

    mesh = plsc.VectorSubcoreMesh(core_axis_name="c", subcore_axis_name="s")

    @functools.partial(
        pl.kernel,
        mesh=mesh,
        out_type=jax.ShapeDtypeStruct((D, B), jnp.float32),
        scratch_types=[
            pltpu.VMEM((b_per_w,), jnp.int32),
            pltpu.VMEM((V * D,), jnp.float32),
            pltpu.VMEM((D, b_per_w), jnp.float32),
            pltpu.SemaphoreType.DMA,
        ],
        compiler_params=pltpu.CompilerParams(needs_layout_passes=False),
    )
    def emb(idx_hbm, table_hbm, out_hbm, idx_v, table_v, buf, ssem):
        wid = lax.axis_index("s") * nc + lax.axis_index("c")
        base = wid * b_per_w
        pltpu.sync_copy(idx_hbm.at[pl.ds(base, b_per_w)], idx_v)
        pltpu.sync_copy(table_hbm, table_v)

        @plsc.parallel_loop(0, b_per_w, step=_LANES, unroll=4)
        def body(i):
            idxv = idx_v[pl.ds(i, _LANES)]
            addr = idxv * D
            for d in range(D):
                v = plsc.load_gather(table_v, [addr + d])
                buf[d, pl.ds(i, _LANES)] = v

        pltpu.async_copy(buf, out_hbm.at[:, pl.ds(base, b_per_w)], ssem).wait()

    def run(world_id, table):
        return emb(world_id, table.reshape(-1)).T

    return run


def kernel(world_id, table):
    B, = world_id.shape
    V, D = table.shape
    return _build(B, V, D)(world_id, table)


# speedup vs baseline: 3.2115x; 1.2719x over previous
import functools

import jax
import jax.numpy as jnp
from jax import lax
from jax.experimental import pallas as pl
from jax.experimental.pallas import tpu as pltpu
from jax.experimental.pallas import tpu_sc as plsc

_LANES = 16


@functools.cache
def _build(B, V, D):
    info = plsc.get_sparse_core_info()
    nc, ns = info.num_cores, info.num_subcores
    nw = nc * ns
    assert B % (nw * _LANES) == 0
    b_per_w = B // nw

    mesh = plsc.VectorSubcoreMesh(core_axis_name="c", subcore_axis_name="s")

    @functools.partial(
        pl.kernel,
        mesh=mesh,
        out_type=jax.ShapeDtypeStruct((D, B), jnp.float32),
        scratch_types=[
            pltpu.VMEM((b_per_w,), jnp.int32),
            pltpu.VMEM((V * D,), jnp.float32),
            pltpu.VMEM((D, b_per_w), jnp.float32),
            pltpu.SemaphoreType.DMA,
        ],
        compiler_params=pltpu.CompilerParams(needs_layout_passes=False),
    )
    def emb(idx_hbm, table_hbm, out_hbm, idx_v, table_v, buf, ssem):
        wid = lax.axis_index("s") * nc + lax.axis_index("c")
        base = wid * b_per_w
        pltpu.sync_copy(idx_hbm.at[pl.ds(base, b_per_w)], idx_v)
        pltpu.sync_copy(table_hbm, table_v)

        @plsc.parallel_loop(0, b_per_w, step=_LANES, unroll=4)
        def body(i):
            idxv = idx_v[pl.ds(i, _LANES)]
            for d in range(D):
                v = plsc.load_gather(table_v, [idxv + d * V])
                buf[d, pl.ds(i, _LANES)] = v

        pltpu.async_copy(buf, out_hbm.at[:, pl.ds(base, b_per_w)], ssem).wait()

    def run(world_id, table):
        return emb(world_id, table.T.reshape(-1)).T

    return run


def kernel(world_id, table):
    B, = world_id.shape
    V, D = table.shape
    return _build(B, V, D)(world_id, table)
